# Initial kernel scaffold; baseline (speedup 1.0000x reference)
import jax
import jax.numpy as jnp
from jax.experimental import pallas as pl


def kernel(pred, target):
    raise NotImplementedError("write your pallas kernel here")


# speedup vs baseline: 12.8519x; 12.8519x over previous
import jax
import jax.numpy as jnp
from jax.experimental import pallas as pl

_CLASSES = 8192
_N_ROWS = 8192
_EPS = 0.1 / (_CLASSES - 2)
_CONF_COEF = 0.9 - _EPS
_ROWS_PER_BLOCK = 128
_GRID = _N_ROWS // _ROWS_PER_BLOCK


def _loss_kernel(pred_ref, tgt_ref, out_ref):
    i = pl.program_id(0)
    block = pred_ref[...]
    t = tgt_ref[0, 0, :]
    m = jnp.max(block, axis=1)
    s = jnp.sum(jnp.exp(block - m[:, None]), axis=1)
    lse = m + jnp.log(s)
    rowsum = jnp.sum(block, axis=1)
    p0 = block[:, 0]
    col = jax.lax.broadcasted_iota(jnp.int32, block.shape, 1)
    pt = jnp.sum(jnp.where(col == t[:, None], block, 0.0), axis=1)
    loss = jnp.where(t != 0, lse - _CONF_COEF * pt - _EPS * (rowsum - p0), 0.0)
    part = jnp.reshape(jnp.sum(loss) * (1.0 / _N_ROWS), (1, 1))

    @pl.when(i == 0)
    def _init():
        out_ref[...] = jnp.zeros((1, 1), jnp.float32)

    out_ref[...] += part


def kernel(pred, target):
    tgt = target.astype(jnp.int32).reshape(_GRID, 1, _ROWS_PER_BLOCK)
    out = pl.pallas_call(
        _loss_kernel,
        grid=(_GRID,),
        in_specs=[
            pl.BlockSpec((_ROWS_PER_BLOCK, _CLASSES), lambda i: (i, 0)),
            pl.BlockSpec((1, 1, _ROWS_PER_BLOCK), lambda i: (i, 0, 0)),
        ],
        out_specs=pl.BlockSpec((1, 1), lambda i: (0, 0)),
        out_shape=jax.ShapeDtypeStruct((1, 1), jnp.float32),
    )(pred, tgt)
    return out[0, 0]


# speedup vs baseline: 16.8548x; 1.3115x over previous
import jax
import jax.numpy as jnp
from jax.experimental import pallas as pl

_CLASSES = 8192
_N_ROWS = 8192
_EPS = 0.1 / (_CLASSES - 2)
_CONF_COEF = 0.9 - _EPS
_ROWS_PER_BLOCK = 512
_GRID = _N_ROWS // _ROWS_PER_BLOCK


def _loss_kernel(pred_ref, tgt_ref, out_ref):
    i = pl.program_id(0)
    block = pred_ref[...]
    t = tgt_ref[0, 0, :]
    m = jnp.max(block, axis=1)
    s = jnp.sum(jnp.exp(block - m[:, None]), axis=1)
    lse = m + jnp.log(s)
    col = jax.lax.broadcasted_iota(jnp.int32, block.shape, 1)
    w = jnp.where(col == t[:, None], 0.9, _EPS)
    wsum = jnp.sum(block * w, axis=1)
    p0 = block[:, 0]
    loss = jnp.where(t != 0, lse - wsum + _EPS * p0, 0.0)
    part = jnp.reshape(jnp.sum(loss) * (1.0 / _N_ROWS), (1, 1))

    @pl.when(i == 0)
    def _init():
        out_ref[...] = jnp.zeros((1, 1), jnp.float32)

    out_ref[...] += part


def kernel(pred, target):
    tgt = target.astype(jnp.int32).reshape(_GRID, 1, _ROWS_PER_BLOCK)
    out = pl.pallas_call(
        _loss_kernel,
        grid=(_GRID,),
        in_specs=[
            pl.BlockSpec((_ROWS_PER_BLOCK, _CLASSES), lambda i: (i, 0)),
            pl.BlockSpec((1, 1, _ROWS_PER_BLOCK), lambda i: (i, 0, 0)),
        ],
        out_specs=pl.BlockSpec((1, 1), lambda i: (0, 0)),
        out_shape=jax.ShapeDtypeStruct((1, 1), jnp.float32),
    )(pred, tgt)
    return out[0, 0]
